# baseline (speedup 1.0000x reference)
import functools

import jax
import jax.numpy as jnp
from jax import lax
from jax.experimental import pallas as pl
from jax.experimental.pallas import tpu as pltpu
from jax.experimental.pallas import tpu_sc as plsc

N = 10000
E = 320000
D = 128
H = 128

NCORE = 2
NSUB = 16
NT = NCORE * NSUB
CH = 128
NCH_A = 79
NCH_B = 79
NCHMAX = max(NCH_A, NCH_B)
NBUF = 2
EPAD = NSUB * (NCH_A + NCH_B) * CH
NPAD = 10240
ROWS_PT = NPAD // NSUB

R = 1000
GRID = N // R

_F32 = jnp.float32


def _sc_body(with_counts, h_ref, pidx_ref, *rest):
    if with_counts:
        out_ref, cnt_ref, pidx, islots, bufs, ones, acc, cacc, gsem, ssem, osem = rest
    else:
        out_ref, pidx, islots, bufs, ones, acc, cacc, gsem, ssem, osem = rest
    c = lax.axis_index("c")
    s = lax.axis_index("s")
    w = c * NSUB + s

    zero16 = jnp.zeros((16,), _F32)
    buf0 = bufs[0]

    @pl.loop(0, CH)
    def _(i):
        for k in range(D // 16):
            buf0[i, pl.ds(k * 16, 16)] = zero16

    if with_counts:
        one16 = jnp.full((16,), 1.0, _F32)
        for k in range(CH // 16):
            ones[pl.ds(k * 16, 16)] = one16

    for k in range(ROWS_PT // CH):
        pltpu.sync_copy(buf0, acc.at[pl.ds(s * ROWS_PT + k * CH, CH)])
        if with_counts:
            pltpu.sync_copy(buf0.at[0], cacc.at[pl.ds(s * ROWS_PT + k * CH, CH)])

    pltpu.sync_copy(pidx_ref.at[w], pidx)
    plsc.subcore_barrier()

    def unpack(j, t):
        isl = islots[t]
        for k in range(CH // 16):
            v = pidx[j, pl.ds(k * 16, 16)]
            isl[0, pl.ds(k * 16, 16)] = lax.bitwise_and(v, 0x3FFF)
            isl[1, pl.ds(k * 16, 16)] = lax.shift_right_logical(v, 14)

    def run(nch):
        gd = [None] * NBUF
        sd = [None] * NBUF
        od = [None] * NBUF
        for t in range(NBUF):
            unpack(t, t)
            gd[t] = pltpu.async_copy(h_ref.at[islots[t].at[0]], bufs[t],
                                     gsem[t])
        for j in range(nch):
            t = j % NBUF
            gd[t].wait()
            sd[t] = pltpu.async_copy(bufs[t], acc.at[islots[t].at[1]],
                                     ssem[t], add=True)
            if with_counts:
                od[t] = pltpu.async_copy(ones, cacc.at[islots[t].at[1]],
                                         osem[t], add=True)
            if j + NBUF < nch:
                sd[t].wait()
                if with_counts:
                    od[t].wait()
                unpack(j + NBUF, t)
                gd[t] = pltpu.async_copy(h_ref.at[islots[t].at[0]], bufs[t],
                                         gsem[t])
        for t in range(NBUF):
            sd[t].wait()
            if with_counts:
                od[t].wait()

    @pl.when(c == 0)
    def _():
        run(NCH_A)

    @pl.when(c == 1)
    def _():
        run(NCH_B)

    plsc.subcore_barrier()
    pltpu.sync_copy(acc.at[pl.ds(s * ROWS_PT, ROWS_PT)],
                    out_ref.at[c, pl.ds(s * ROWS_PT, ROWS_PT)])
    if with_counts:
        pltpu.sync_copy(cacc.at[pl.ds(s * ROWS_PT, ROWS_PT)],
                        cnt_ref.at[c, pl.ds(s * ROWS_PT, ROWS_PT)])


def _sc_segsum(h, pidx, with_counts):
    mesh = plsc.VectorSubcoreMesh(core_axis_name="c", subcore_axis_name="s")
    if with_counts:
        out_type = [jax.ShapeDtypeStruct((NCORE, NPAD, D), _F32),
                    jax.ShapeDtypeStruct((NCORE, NPAD), _F32)]
    else:
        out_type = jax.ShapeDtypeStruct((NCORE, NPAD, D), _F32)
    scratch = [
        pltpu.VMEM((NCHMAX, CH), jnp.int32),
        [pltpu.VMEM((2, CH), jnp.int32)] * NBUF,
        [pltpu.VMEM((CH, D), _F32)] * NBUF,
        pltpu.VMEM((CH,), _F32),
        pltpu.VMEM_SHARED((NPAD, D), _F32),
        pltpu.VMEM_SHARED((NPAD,), _F32),
        [pltpu.SemaphoreType.DMA] * NBUF,
        [pltpu.SemaphoreType.DMA] * NBUF,
        [pltpu.SemaphoreType.DMA] * NBUF,
    ]
    return pl.kernel(
        functools.partial(_sc_body, with_counts),
        out_type,
        mesh=mesh,
        scratch_types=scratch,
    )(h, pidx)


def _gelu(v):
    return 0.5 * v * (1.0 + lax.erf(v * 0.7071067811865476))


def _tc1_body(x, s0a, s0b, ca, cb, Wp, bp, Wl0, bl0, Wr0, h1_out):
    cnt = jnp.maximum(ca[...] + cb[...], 1.0)
    mean = (s0a[0] + s0b[0]) / cnt
    pre = (jnp.dot(mean, Wl0[...], preferred_element_type=_F32) + bl0[...]
           + jnp.dot(x[...], Wr0[...], preferred_element_type=_F32))
    res = jnp.dot(x[...], Wp[...], preferred_element_type=_F32) + bp[...]
    h1_out[...] = _gelu(pre) + res


def _tc1(x, s0, ca, cb, Wp, bp, Wl0, bl0, Wr0):
    full = lambda shape: pl.BlockSpec(shape, lambda i: tuple(0 for _ in shape))
    return pl.pallas_call(
        _tc1_body,
        grid=(GRID,),
        in_specs=[
            pl.BlockSpec((R, D), lambda i: (i, 0)),
            pl.BlockSpec((1, R, D), lambda i: (0, i, 0)),
            pl.BlockSpec((1, R, D), lambda i: (1, i, 0)),
            pl.BlockSpec((R, 1), lambda i: (i, 0)),
            pl.BlockSpec((R, 1), lambda i: (i, 0)),
            full((D, H)), full((1, H)),
            full((D, H)), full((1, H)),
            full((D, H)),
        ],
        out_specs=pl.BlockSpec((R, H), lambda i: (i, 0)),
        out_shape=jax.ShapeDtypeStruct((N, H), _F32),
    )(x, s0, s0, ca, cb, Wp, bp, Wl0, bl0, Wr0)


def _tc2_body(h1, s1a, s1b, ca, cb, Wl1, bl1, Wr1, W1, b1, w2r, b2, rer,
              alpha, out):
    cnt = jnp.maximum(ca[...] + cb[...], 1.0)
    mean = (s1a[0] + s1b[0]) / cnt
    pre = (jnp.dot(mean, Wl1[...], preferred_element_type=_F32) + bl1[...]
           + jnp.dot(h1[...], Wr1[...], preferred_element_type=_F32))
    h2 = _gelu(pre) + h1[...]
    u = _gelu(jnp.dot(h2, W1[...], preferred_element_type=_F32) + b1[...])
    gnn = jnp.sum(u * w2r[...], axis=1, keepdims=True) + b2[...]
    a = alpha[...]
    out[...] = a * rer[...] + (1.0 - a) * gnn


def _tc2(h1, s1, ca, cb, Wl1, bl1, Wr1, W1, b1, w2r, b2, rer, alpha):
    full = lambda shape: pl.BlockSpec(shape, lambda i: tuple(0 for _ in shape))
    return pl.pallas_call(
        _tc2_body,
        grid=(GRID,),
        in_specs=[
            pl.BlockSpec((R, H), lambda i: (i, 0)),
            pl.BlockSpec((1, R, H), lambda i: (0, i, 0)),
            pl.BlockSpec((1, R, H), lambda i: (1, i, 0)),
            pl.BlockSpec((R, 1), lambda i: (i, 0)),
            pl.BlockSpec((R, 1), lambda i: (i, 0)),
            full((H, H)), full((1, H)),
            full((H, H)),
            full((H, H // 2)), full((1, H // 2)),
            full((1, H // 2)), full((1, 1)),
            pl.BlockSpec((R, 1), lambda i: (i, 0)),
            full((1, 1)),
        ],
        out_specs=pl.BlockSpec((R, 1), lambda i: (i, 0)),
        out_shape=jax.ShapeDtypeStruct((N, 1), _F32),
    )(h1, s1, s1, ca, cb, Wl1, bl1, Wr1, W1, b1, w2r, b2, rer, alpha)


def kernel(x, edge_index, reranker_scores, Wp, bp, Wl0, bl0, Wr0, Wl1, bl1,
           Wr1, W1, b1, W2, b2, alpha_logit):
    src = edge_index[0]
    dst = edge_index[1]
    pad = EPAD - E
    srcp = jnp.concatenate([src, jnp.zeros((pad,), jnp.int32)])
    pad_dst = N + jnp.arange(pad, dtype=jnp.int32) % (NPAD - N)
    dstp = jnp.concatenate([dst, pad_dst])
    ij_flat = jnp.bitwise_or(srcp, jnp.left_shift(dstp, 14))
    na = NSUB * NCH_A * CH
    ij_a = ij_flat[:na].reshape(NSUB, NCH_A, CH)
    ij_b = ij_flat[na:].reshape(NSUB, NCH_B, CH)
    if NCH_A < NCH_B:
        ij_a = jnp.pad(ij_a, ((0, 0), (0, NCHMAX - NCH_A), (0, 0)))
    elif NCH_B < NCH_A:
        ij_b = jnp.pad(ij_b, ((0, 0), (0, NCHMAX - NCH_B), (0, 0)))
    pidx = jnp.concatenate([ij_a, ij_b], axis=0)

    s0, cnt = _sc_segsum(x, pidx, with_counts=True)
    ca = cnt[0, :N].reshape(N, 1)
    cb = cnt[1, :N].reshape(N, 1)

    h1 = _tc1(x, s0, ca, cb, Wp, bp.reshape(1, H), Wl0, bl0.reshape(1, H), Wr0)

    s1 = _sc_segsum(h1, pidx, with_counts=False)

    alpha = jax.nn.sigmoid(alpha_logit).reshape(1, 1)
    out = _tc2(h1, s1, ca, cb, Wl1, bl1.reshape(1, H), Wr1,
               W1, b1.reshape(1, H // 2), W2.reshape(1, H // 2),
               b2.reshape(1, 1), reranker_scores.reshape(N, 1), alpha)
    return out.reshape(N)


# speedup vs baseline: 1.0009x; 1.0009x over previous
import functools

import jax
import jax.numpy as jnp
from jax import lax
from jax.experimental import pallas as pl
from jax.experimental.pallas import tpu as pltpu
from jax.experimental.pallas import tpu_sc as plsc

N = 10000
E = 320000
D = 128
H = 128

NCORE = 2
NSUB = 16
NT = NCORE * NSUB
CH = 128
NCH = 79
NBUF = 2
EPAD = NT * NCH * CH
NPAD = 10240
ROWS_PT = NPAD // NSUB

R = 1000
GRID = N // R

_F32 = jnp.float32


def _sc_body(with_counts, h_ref, pidx_ref, *rest):
    if with_counts:
        out_ref, cnt_ref, pidx, islots, bufs, ones, acc, cacc, gsem, ssem, osem = rest
    else:
        out_ref, pidx, islots, bufs, ones, acc, cacc, gsem, ssem, osem = rest
    c = lax.axis_index("c")
    s = lax.axis_index("s")
    w = c * NSUB + s

    zero16 = jnp.zeros((16,), _F32)
    buf0 = bufs[0]

    @pl.loop(0, CH)
    def _(i):
        for k in range(D // 16):
            buf0[i, pl.ds(k * 16, 16)] = zero16

    if with_counts:
        one16 = jnp.full((16,), 1.0, _F32)
        for k in range(CH // 16):
            ones[pl.ds(k * 16, 16)] = one16

    for k in range(ROWS_PT // CH):
        pltpu.sync_copy(buf0, acc.at[pl.ds(s * ROWS_PT + k * CH, CH)])
        if with_counts:
            pltpu.sync_copy(buf0.at[0], cacc.at[pl.ds(s * ROWS_PT + k * CH, CH)])

    pltpu.sync_copy(pidx_ref.at[w], pidx)
    plsc.subcore_barrier()

    def unpack(j, t):
        isl = islots[t]
        for k in range(CH // 16):
            v = pidx[j, pl.ds(k * 16, 16)]
            isl[0, pl.ds(k * 16, 16)] = lax.bitwise_and(v, 0x3FFF)
            isl[1, pl.ds(k * 16, 16)] = lax.shift_right_logical(v, 14)

    gd = [None] * NBUF
    sd = [None] * NBUF
    od = [None] * NBUF
    for t in range(NBUF):
        unpack(t, t)
        gd[t] = pltpu.async_copy(h_ref.at[islots[t].at[0]], bufs[t], gsem[t])
    for j in range(NCH):
        t = j % NBUF
        gd[t].wait()
        sd[t] = pltpu.async_copy(bufs[t], acc.at[islots[t].at[1]],
                                 ssem[t], add=True)
        if with_counts:
            od[t] = pltpu.async_copy(ones, cacc.at[islots[t].at[1]],
                                     osem[t], add=True)
        if j + NBUF < NCH:
            sd[t].wait()
            if with_counts:
                od[t].wait()
            unpack(j + NBUF, t)
            gd[t] = pltpu.async_copy(h_ref.at[islots[t].at[0]], bufs[t],
                                     gsem[t])
    for t in range(NBUF):
        sd[t].wait()
        if with_counts:
            od[t].wait()

    plsc.subcore_barrier()
    pltpu.sync_copy(acc.at[pl.ds(s * ROWS_PT, ROWS_PT)],
                    out_ref.at[c, pl.ds(s * ROWS_PT, ROWS_PT)])
    if with_counts:
        pltpu.sync_copy(cacc.at[pl.ds(s * ROWS_PT, ROWS_PT)],
                        cnt_ref.at[c, pl.ds(s * ROWS_PT, ROWS_PT)])


def _sc_segsum(h, pidx, with_counts):
    mesh = plsc.VectorSubcoreMesh(core_axis_name="c", subcore_axis_name="s")
    if with_counts:
        out_type = [jax.ShapeDtypeStruct((NCORE, NPAD, D), _F32),
                    jax.ShapeDtypeStruct((NCORE, NPAD), _F32)]
    else:
        out_type = jax.ShapeDtypeStruct((NCORE, NPAD, D), _F32)
    scratch = [
        pltpu.VMEM((NCH, CH), jnp.int32),
        [pltpu.VMEM((2, CH), jnp.int32)] * NBUF,
        [pltpu.VMEM((CH, D), _F32)] * NBUF,
        pltpu.VMEM((CH,), _F32),
        pltpu.VMEM_SHARED((NPAD, D), _F32),
        pltpu.VMEM_SHARED((NPAD,), _F32),
        [pltpu.SemaphoreType.DMA] * NBUF,
        [pltpu.SemaphoreType.DMA] * NBUF,
        [pltpu.SemaphoreType.DMA] * NBUF,
    ]
    return pl.kernel(
        functools.partial(_sc_body, with_counts),
        out_type,
        mesh=mesh,
        scratch_types=scratch,
    )(h, pidx)


def _gelu(v):
    return 0.5 * v * (1.0 + lax.erf(v * 0.7071067811865476))


def _tc1_body(x, s0a, s0b, ca, cb, Wp, bp, Wl0, bl0, Wr0, h1_out):
    cnt = jnp.maximum(ca[...] + cb[...], 1.0)
    mean = (s0a[0] + s0b[0]) / cnt
    pre = (jnp.dot(mean, Wl0[...], preferred_element_type=_F32) + bl0[...]
           + jnp.dot(x[...], Wr0[...], preferred_element_type=_F32))
    res = jnp.dot(x[...], Wp[...], preferred_element_type=_F32) + bp[...]
    h1_out[...] = _gelu(pre) + res


def _tc1(x, s0, ca, cb, Wp, bp, Wl0, bl0, Wr0):
    full = lambda shape: pl.BlockSpec(shape, lambda i: tuple(0 for _ in shape))
    return pl.pallas_call(
        _tc1_body,
        grid=(GRID,),
        in_specs=[
            pl.BlockSpec((R, D), lambda i: (i, 0)),
            pl.BlockSpec((1, R, D), lambda i: (0, i, 0)),
            pl.BlockSpec((1, R, D), lambda i: (1, i, 0)),
            pl.BlockSpec((R, 1), lambda i: (i, 0)),
            pl.BlockSpec((R, 1), lambda i: (i, 0)),
            full((D, H)), full((1, H)),
            full((D, H)), full((1, H)),
            full((D, H)),
        ],
        out_specs=pl.BlockSpec((R, H), lambda i: (i, 0)),
        out_shape=jax.ShapeDtypeStruct((N, H), _F32),
    )(x, s0, s0, ca, cb, Wp, bp, Wl0, bl0, Wr0)


def _tc2_body(h1, s1a, s1b, ca, cb, Wl1, bl1, Wr1, W1, b1, w2r, b2, rer,
              alpha, out):
    cnt = jnp.maximum(ca[...] + cb[...], 1.0)
    mean = (s1a[0] + s1b[0]) / cnt
    pre = (jnp.dot(mean, Wl1[...], preferred_element_type=_F32) + bl1[...]
           + jnp.dot(h1[...], Wr1[...], preferred_element_type=_F32))
    h2 = _gelu(pre) + h1[...]
    u = _gelu(jnp.dot(h2, W1[...], preferred_element_type=_F32) + b1[...])
    gnn = jnp.sum(u * w2r[...], axis=1, keepdims=True) + b2[...]
    a = alpha[...]
    out[...] = a * rer[...] + (1.0 - a) * gnn


def _tc2(h1, s1, ca, cb, Wl1, bl1, Wr1, W1, b1, w2r, b2, rer, alpha):
    full = lambda shape: pl.BlockSpec(shape, lambda i: tuple(0 for _ in shape))
    return pl.pallas_call(
        _tc2_body,
        grid=(GRID,),
        in_specs=[
            pl.BlockSpec((R, H), lambda i: (i, 0)),
            pl.BlockSpec((1, R, H), lambda i: (0, i, 0)),
            pl.BlockSpec((1, R, H), lambda i: (1, i, 0)),
            pl.BlockSpec((R, 1), lambda i: (i, 0)),
            pl.BlockSpec((R, 1), lambda i: (i, 0)),
            full((H, H)), full((1, H)),
            full((H, H)),
            full((H, H // 2)), full((1, H // 2)),
            full((1, H // 2)), full((1, 1)),
            pl.BlockSpec((R, 1), lambda i: (i, 0)),
            full((1, 1)),
        ],
        out_specs=pl.BlockSpec((R, 1), lambda i: (i, 0)),
        out_shape=jax.ShapeDtypeStruct((N, 1), _F32),
    )(h1, s1, s1, ca, cb, Wl1, bl1, Wr1, W1, b1, w2r, b2, rer, alpha)


def kernel(x, edge_index, reranker_scores, Wp, bp, Wl0, bl0, Wr0, Wl1, bl1,
           Wr1, W1, b1, W2, b2, alpha_logit):
    src = edge_index[0]
    dst = edge_index[1]
    pad = EPAD - E
    srcp = jnp.concatenate([src, jnp.zeros((pad,), jnp.int32)])
    pad_dst = N + jnp.arange(pad, dtype=jnp.int32) % (NPAD - N)
    dstp = jnp.concatenate([dst, pad_dst])
    pidx = jnp.bitwise_or(srcp, jnp.left_shift(dstp, 14)).reshape(NT, NCH, CH)

    s0, cnt = _sc_segsum(x, pidx, with_counts=True)
    ca = cnt[0, :N].reshape(N, 1)
    cb = cnt[1, :N].reshape(N, 1)

    h1 = _tc1(x, s0, ca, cb, Wp, bp.reshape(1, H), Wl0, bl0.reshape(1, H), Wr0)

    s1 = _sc_segsum(h1, pidx, with_counts=False)

    alpha = jax.nn.sigmoid(alpha_logit).reshape(1, 1)
    out = _tc2(h1, s1, ca, cb, Wl1, bl1.reshape(1, H), Wr1,
               W1, b1.reshape(1, H // 2), W2.reshape(1, H // 2),
               b2.reshape(1, 1), reranker_scores.reshape(N, 1), alpha)
    return out.reshape(N)
